# baseline (speedup 1.0000x reference)
import functools

import jax
import jax.numpy as jnp
from jax import lax
from jax.experimental import pallas as pl
from jax.experimental.pallas import tpu as pltpu
from jax.experimental.pallas import tpu_sc as plsc

_BATCH = 16384
_N_FIELDS = 26
_DIM = 64
_VOCAB = 1000000

_NC = 2
_NS = 16
_NW = _NC * _NS

_CBLK = 128
_NBLK = _VOCAB // _CBLK
_TAIL = _VOCAB - _NBLK * _CBLK
_LANES16 = 16


def _transpose_body(embT_hbm, tailT_hbm, flat_hbm, in_v, tail_v, out_v,
                    isems, osems):
    wid = lax.axis_index("s") * _NC + lax.axis_index("c")
    lanes = lax.iota(jnp.int32, _LANES16)
    nblk = (_NBLK - 1 - wid) // _NW + 1

    def start_load(b, t):
        bk = wid + t * _NW
        col0 = pl.multiple_of(bk * _CBLK, _CBLK)
        pltpu.async_copy(
            embT_hbm.at[:, pl.ds(col0, _CBLK)], in_v.at[b], isems.at[b]
        )

    def wait_load(b):
        pltpu.make_async_copy(
            embT_hbm.at[:, pl.ds(0, _CBLK)], in_v.at[b], isems.at[b]
        ).wait()

    def start_store(b, t):
        bk = wid + t * _NW
        off = bk * (_CBLK * _DIM)
        pltpu.async_copy(
            out_v.at[pl.ds(b * (_CBLK * _DIM), _CBLK * _DIM)],
            flat_hbm.at[pl.ds(off, _CBLK * _DIM)],
            osems.at[b],
        )

    def wait_store(b):
        pltpu.make_async_copy(
            out_v.at[pl.ds(b * (_CBLK * _DIM), _CBLK * _DIM)],
            flat_hbm.at[pl.ds(0, _CBLK * _DIM)],
            osems.at[b],
        ).wait()

    def transpose_block(src, nrows, b):
        mask = nrows - 1

        base = b * (_CBLK * _DIM)

        @plsc.parallel_loop(0, nrows, step=1, unroll=8)
        def _(j):
            jw = jnp.bitwise_and(j + lanes, mask)
            jw64 = base + jw * _DIM
            for rr in range(_DIM // _LANES16):
                dvec = rr * _LANES16 + lanes
                v = plsc.load_gather(src, [dvec, jw])
                plsc.store_scatter(out_v, [jw64 + dvec], v)

    start_load(0, 0)

    def step2(t2, carry):
        for b in range(2):
            t = t2 * 2 + b
            wait_load(b)

            @pl.when(t + 1 < nblk)
            def _():
                start_load(1 - b, t + 1)

            @pl.when(t >= 2)
            def _():
                wait_store(b)

            transpose_block(in_v.at[b], _CBLK, b)
            start_store(b, t)
        return carry

    lax.fori_loop(0, nblk // 2, step2, 0)

    @pl.when(lax.rem(nblk, 2) == 1)
    def _():
        t = nblk - 1
        wait_load(0)
        wait_store(0)
        transpose_block(in_v.at[0], _CBLK, 0)
        start_store(0, t)

    @pl.when(wid == 0)
    def _():
        pltpu.sync_copy(tailT_hbm, tail_v)
        wait_store(lax.rem(nblk, 2))
        transpose_block(tail_v, _TAIL, lax.rem(nblk, 2))
        pltpu.async_copy(
            out_v.at[pl.ds(lax.rem(nblk, 2) * (_CBLK * _DIM), _TAIL * _DIM)],
            flat_hbm.at[pl.ds(_NBLK * _CBLK * _DIM, _TAIL * _DIM)],
            osems.at[lax.rem(nblk, 2)],
        )
        pltpu.make_async_copy(
            out_v.at[pl.ds(lax.rem(nblk, 2) * (_CBLK * _DIM), _TAIL * _DIM)],
            flat_hbm.at[pl.ds(0, _TAIL * _DIM)],
            osems.at[lax.rem(nblk, 2)],
        ).wait()

    @pl.when(wid != 0)
    def _():
        wait_store(lax.rem(nblk, 2))
    wait_store(lax.rem(nblk + 1, 2))


@functools.partial(
    pl.kernel,
    mesh=plsc.VectorSubcoreMesh(core_axis_name="c", subcore_axis_name="s"),
    out_type=jax.ShapeDtypeStruct((_VOCAB * _DIM,), jnp.float32),
    scratch_types=[
        pltpu.VMEM((2, _DIM, _CBLK), jnp.float32),
        pltpu.VMEM((_DIM, _TAIL), jnp.float32),
        pltpu.VMEM((2 * _CBLK * _DIM,), jnp.float32),
        pltpu.SemaphoreType.DMA((2,)),
        pltpu.SemaphoreType.DMA((2,)),
    ],
    compiler_params=pltpu.CompilerParams(
        use_tc_tiling_on_sc=True, needs_layout_passes=False
    ),
)
def _transpose_call(embT_hbm, tailT_hbm, flat_hbm, in_v, tail_v, out_v,
                    isems, osems):
    _transpose_body(embT_hbm, tailT_hbm, flat_hbm, in_v, tail_v, out_v,
                    isems, osems)


_B_PER_W = _BATCH // _NW
_NBUF = 2
_N_PAIRS = _N_FIELDS // _NBUF


def _gather_body(idxT_hbm, table_hbm, out_hbm, idx_v, rows_v, gsems, wsems):
    wid = lax.axis_index("s") * _NC + lax.axis_index("c")
    b0 = wid * _B_PER_W

    def start_gather(b, f):
        pltpu.async_copy(
            table_hbm.at[idx_v.at[f]], rows_v.at[b], gsems.at[b]
        )

    def wait_gather(b, f):
        pltpu.make_async_copy(
            table_hbm.at[idx_v.at[f]], rows_v.at[b], gsems.at[b]
        ).wait()

    def start_writeback(b, f):
        pltpu.async_copy(
            rows_v.at[b], out_hbm.at[pl.ds(b0, _B_PER_W), f], wsems.at[b]
        )

    def wait_writeback(b):
        pltpu.make_async_copy(
            rows_v.at[b], out_hbm.at[pl.ds(b0, _B_PER_W), 0], wsems.at[b]
        ).wait()

    pltpu.sync_copy(idxT_hbm.at[:, pl.ds(b0, _B_PER_W)], idx_v)

    for b in range(_NBUF):
        start_gather(b, b)

    def pair(j, carry):
        for b in range(_NBUF):
            f = j * _NBUF + b
            wait_gather(b, f)
            start_writeback(b, f)

            @pl.when(j < _N_PAIRS - 1)
            def _():
                wait_writeback(b)
                start_gather(b, f + _NBUF)

        return carry

    lax.fori_loop(0, _N_PAIRS, pair, 0)

    for b in range(_NBUF):
        wait_writeback(b)


@functools.partial(
    pl.kernel,
    mesh=plsc.VectorSubcoreMesh(core_axis_name="c", subcore_axis_name="s"),
    out_type=jax.ShapeDtypeStruct((_BATCH, _N_FIELDS, _DIM), jnp.float32),
    scratch_types=[
        pltpu.VMEM((_N_FIELDS, _B_PER_W), jnp.int32),
        pltpu.VMEM((_NBUF, _B_PER_W, _DIM), jnp.float32),
        pltpu.SemaphoreType.DMA((_NBUF,)),
        pltpu.SemaphoreType.DMA((_NBUF,)),
    ],
    compiler_params=pltpu.CompilerParams(use_tc_tiling_on_sc=False),
)
def _gather_call(idxT_hbm, table_hbm, out_hbm, idx_v, rows_v, gsems, wsems):
    _gather_body(idxT_hbm, table_hbm, out_hbm, idx_v, rows_v, gsems, wsems)


@jax.jit
def kernel(X, embeddings):
    embT = embeddings.T
    flat = _transpose_call(embT, embT[:, _NBLK * _CBLK:])
    table = flat.reshape(_VOCAB, _DIM)
    return _gather_call(X.T.astype(jnp.int32), table)


# speedup vs baseline: 1.1567x; 1.1567x over previous
import functools

import jax
import jax.numpy as jnp
from jax import lax
from jax.experimental import pallas as pl
from jax.experimental.pallas import tpu as pltpu
from jax.experimental.pallas import tpu_sc as plsc

_BATCH = 16384
_N_FIELDS = 26
_DIM = 64
_VOCAB = 1000000

_NC = 2
_NS = 16
_NW = _NC * _NS

_CBLK = 256
_NBLK = _VOCAB // _CBLK
_TAIL = _VOCAB - _NBLK * _CBLK
_LANES16 = 16


def _transpose_body(embT_hbm, tailT_hbm, flat_hbm, in_v, tail_v, out_v,
                    isems, osems):
    wid = lax.axis_index("s") * _NC + lax.axis_index("c")
    lanes = lax.iota(jnp.int32, _LANES16)
    nblk = (_NBLK - 1 - wid) // _NW + 1

    def start_load(b, t):
        bk = wid + t * _NW
        col0 = pl.multiple_of(bk * _CBLK, _CBLK)
        pltpu.async_copy(
            embT_hbm.at[:, pl.ds(col0, _CBLK)], in_v.at[b], isems.at[b]
        )

    def wait_load(b):
        pltpu.make_async_copy(
            embT_hbm.at[:, pl.ds(0, _CBLK)], in_v.at[b], isems.at[b]
        ).wait()

    def start_store(b, t):
        bk = wid + t * _NW
        off = bk * (_CBLK * _DIM)
        pltpu.async_copy(
            out_v.at[pl.ds(b * (_CBLK * _DIM), _CBLK * _DIM)],
            flat_hbm.at[pl.ds(off, _CBLK * _DIM)],
            osems.at[b],
        )

    def wait_store(b):
        pltpu.make_async_copy(
            out_v.at[pl.ds(b * (_CBLK * _DIM), _CBLK * _DIM)],
            flat_hbm.at[pl.ds(0, _CBLK * _DIM)],
            osems.at[b],
        ).wait()

    def transpose_block(src, nrows, b):
        mask = nrows - 1

        base = b * (_CBLK * _DIM)

        @plsc.parallel_loop(0, nrows, step=1, unroll=4)
        def _(j):
            jw = jnp.bitwise_and(j + lanes, mask)
            jw64 = base + jw * _DIM
            for rr in range(_DIM // _LANES16):
                dvec = rr * _LANES16 + lanes
                v = plsc.load_gather(src, [dvec, jw])
                plsc.store_scatter(out_v, [jw64 + dvec], v)

    start_load(0, 0)

    def step2(t2, carry):
        for b in range(2):
            t = t2 * 2 + b
            wait_load(b)

            @pl.when(t + 1 < nblk)
            def _():
                start_load(1 - b, t + 1)

            @pl.when(t >= 2)
            def _():
                wait_store(b)

            transpose_block(in_v.at[b], _CBLK, b)
            start_store(b, t)
        return carry

    lax.fori_loop(0, nblk // 2, step2, 0)

    @pl.when(lax.rem(nblk, 2) == 1)
    def _():
        t = nblk - 1
        wait_load(0)
        wait_store(0)
        transpose_block(in_v.at[0], _CBLK, 0)
        start_store(0, t)

    @pl.when(wid == 0)
    def _():
        pltpu.sync_copy(tailT_hbm, tail_v)
        wait_store(lax.rem(nblk, 2))
        transpose_block(tail_v, _TAIL, lax.rem(nblk, 2))
        pltpu.async_copy(
            out_v.at[pl.ds(lax.rem(nblk, 2) * (_CBLK * _DIM), _TAIL * _DIM)],
            flat_hbm.at[pl.ds(_NBLK * _CBLK * _DIM, _TAIL * _DIM)],
            osems.at[lax.rem(nblk, 2)],
        )
        pltpu.make_async_copy(
            out_v.at[pl.ds(lax.rem(nblk, 2) * (_CBLK * _DIM), _TAIL * _DIM)],
            flat_hbm.at[pl.ds(0, _TAIL * _DIM)],
            osems.at[lax.rem(nblk, 2)],
        ).wait()

    @pl.when(wid != 0)
    def _():
        wait_store(lax.rem(nblk, 2))
    wait_store(lax.rem(nblk + 1, 2))


@functools.partial(
    pl.kernel,
    mesh=plsc.VectorSubcoreMesh(core_axis_name="c", subcore_axis_name="s"),
    out_type=jax.ShapeDtypeStruct((_VOCAB * _DIM,), jnp.float32),
    scratch_types=[
        pltpu.VMEM((2, _DIM, _CBLK), jnp.float32),
        pltpu.VMEM((_DIM, _TAIL), jnp.float32),
        pltpu.VMEM((2 * _CBLK * _DIM,), jnp.float32),
        pltpu.SemaphoreType.DMA((2,)),
        pltpu.SemaphoreType.DMA((2,)),
    ],
    compiler_params=pltpu.CompilerParams(
        use_tc_tiling_on_sc=True, needs_layout_passes=False
    ),
)
def _transpose_call(embT_hbm, tailT_hbm, flat_hbm, in_v, tail_v, out_v,
                    isems, osems):
    _transpose_body(embT_hbm, tailT_hbm, flat_hbm, in_v, tail_v, out_v,
                    isems, osems)


_B_PER_W = _BATCH // _NW
_NBUF = 2
_N_PAIRS = _N_FIELDS // _NBUF


def _gather_body(idxT_hbm, table_hbm, out_hbm, idx_v, rows_v, gsems, wsems):
    wid = lax.axis_index("s") * _NC + lax.axis_index("c")
    b0 = wid * _B_PER_W

    def start_gather(b, f):
        pltpu.async_copy(
            table_hbm.at[idx_v.at[f]], rows_v.at[b], gsems.at[b]
        )

    def wait_gather(b, f):
        pltpu.make_async_copy(
            table_hbm.at[idx_v.at[f]], rows_v.at[b], gsems.at[b]
        ).wait()

    def start_writeback(b, f):
        pltpu.async_copy(
            rows_v.at[b], out_hbm.at[pl.ds(b0, _B_PER_W), f], wsems.at[b]
        )

    def wait_writeback(b):
        pltpu.make_async_copy(
            rows_v.at[b], out_hbm.at[pl.ds(b0, _B_PER_W), 0], wsems.at[b]
        ).wait()

    pltpu.sync_copy(idxT_hbm.at[:, pl.ds(b0, _B_PER_W)], idx_v)

    for b in range(_NBUF):
        start_gather(b, b)

    def pair(j, carry):
        for b in range(_NBUF):
            f = j * _NBUF + b
            wait_gather(b, f)
            start_writeback(b, f)

            @pl.when(j < _N_PAIRS - 1)
            def _():
                wait_writeback(b)
                start_gather(b, f + _NBUF)

        return carry

    lax.fori_loop(0, _N_PAIRS, pair, 0)

    for b in range(_NBUF):
        wait_writeback(b)


@functools.partial(
    pl.kernel,
    mesh=plsc.VectorSubcoreMesh(core_axis_name="c", subcore_axis_name="s"),
    out_type=jax.ShapeDtypeStruct((_BATCH, _N_FIELDS, _DIM), jnp.float32),
    scratch_types=[
        pltpu.VMEM((_N_FIELDS, _B_PER_W), jnp.int32),
        pltpu.VMEM((_NBUF, _B_PER_W, _DIM), jnp.float32),
        pltpu.SemaphoreType.DMA((_NBUF,)),
        pltpu.SemaphoreType.DMA((_NBUF,)),
    ],
    compiler_params=pltpu.CompilerParams(use_tc_tiling_on_sc=False),
)
def _gather_call(idxT_hbm, table_hbm, out_hbm, idx_v, rows_v, gsems, wsems):
    _gather_body(idxT_hbm, table_hbm, out_hbm, idx_v, rows_v, gsems, wsems)


@jax.jit
def kernel(X, embeddings):
    embT = embeddings.T
    flat = _transpose_call(embT, embT[:, _NBLK * _CBLK:])
    table = flat.reshape(_VOCAB, _DIM)
    return _gather_call(X.T.astype(jnp.int32), table)
